# baseline (speedup 1.0000x reference)
import functools

import jax
import jax.numpy as jnp
from jax import lax
from jax.experimental import pallas as pl
from jax.experimental.pallas import tpu as pltpu
from jax.experimental.pallas import tpu_sc as plsc

B, C, N, Q, K = 8, 256, 2048, 64, 1024
NB = 512
BN = B * N
NW = 32
CHUNK = BN // NW


def _tc_body(qp_ref, ze_ref, emb_ref, md_ref, out_ref):
    i = pl.program_id(0)

    emb_v = emb_ref[...]
    emb_sq = jnp.sum(emb_v * emb_v, axis=1)
    ze_v = ze_ref[...]
    ze_sq = jnp.sum(ze_v * ze_v, axis=1)

    acc = jnp.float32(0.0)
    for b in range(B):
        cross = jnp.dot(emb_v, ze_v[b],
                        preferred_element_type=jnp.float32)
        d = emb_sq[:, None] - 2.0 * cross
        acc += jnp.sum(jnp.min(d, axis=0))
    acc += jnp.sum(ze_sq)

    x = qp_ref[...]
    mx = jnp.max(x, axis=1)
    lse = jnp.log(jnp.sum(jnp.exp(x - mx[:, None, :]), axis=1)) + mx
    acc += jnp.sum(lse)

    md_sum = jnp.sum(md_ref[...])

    @pl.when(i == 0)
    def _():
        out_ref[...] = jnp.zeros_like(out_ref)

    out_ref[0, :] += jnp.broadcast_to(acc, (128,))
    out_ref[1, :] += jnp.broadcast_to(md_sum, (128,))


def _tc_call(quant_pred, ze, emb, min_dist):
    return pl.pallas_call(
        _tc_body,
        grid=(N // NB,),
        in_specs=[
            pl.BlockSpec((B, C, NB), lambda i: (0, 0, i)),
            pl.BlockSpec((B, Q, NB), lambda i: (0, 0, i)),
            pl.BlockSpec((K, Q), lambda i: (0, 0)),
            pl.BlockSpec((B, NB), lambda i: (0, i)),
        ],
        out_specs=pl.BlockSpec((2, 128), lambda i: (0, 0)),
        out_shape=jax.ShapeDtypeStruct((2, 128), jnp.float32),
    )(quant_pred, ze, emb, min_dist)


def _sc_gather_sum(qp_flat, tgt_flat):
    mesh = plsc.VectorSubcoreMesh(core_axis_name="c", subcore_axis_name="s")

    @functools.partial(
        pl.kernel,
        mesh=mesh,
        out_type=jax.ShapeDtypeStruct((NW, 16), jnp.float32),
        scratch_types=[
            pltpu.VMEM((CHUNK,), jnp.int32),
            pltpu.VMEM((4, 128), jnp.int32),
            pltpu.VMEM((4, 128), jnp.float32),
            pltpu.VMEM((16,), jnp.float32),
            pltpu.SemaphoreType.DMA,
        ],
    )
    def k(qp_hbm, tgt_hbm, out_hbm, tgt_v, idx_v, val_v, acc_v, sem):
        wid = lax.axis_index("s") * 2 + lax.axis_index("c")
        base = wid * CHUNK
        b = base // N
        n0 = base - b * N

        pltpu.sync_copy(tgt_hbm.at[pl.ds(base, CHUNK)], tgt_v)

        lane = lax.broadcasted_iota(jnp.int32, (16,), 0)
        bCN = b * (C * N)
        for j in range(CHUNK // 16):
            t16 = tgt_v[pl.ds(j * 16, 16)]
            idx16 = bCN + t16 * N + (n0 + j * 16) + lane
            idx_v[j // 8, pl.ds((j % 8) * 16, 16)] = idx16

        cps = [
            pltpu.async_copy(qp_hbm.at[idx_v.at[r]], val_v.at[r], sem)
            for r in range(4)
        ]
        for cp in cps:
            cp.wait()

        acc = jnp.zeros((16,), jnp.float32)
        for r in range(4):
            for c8 in range(8):
                acc = acc + val_v[r, pl.ds(c8 * 16, 16)]
        acc_v[...] = acc
        pltpu.sync_copy(acc_v, out_hbm.at[wid])

    return k(qp_flat, tgt_flat)


def kernel(quant_pred, target_wav, ze, emb, min_dist, gamma=0.25):
    tgt_flat = target_wav.astype(jnp.int32).reshape(BN)
    qp_flat = quant_pred.reshape(B * C * N)
    sc_part = _sc_gather_sum(qp_flat, tgt_flat)
    tc_part = _tc_call(quant_pred, ze, emb, min_dist)
    return tc_part[0, 0] - jnp.sum(sc_part) + gamma * tc_part[1, 0]


# speedup vs baseline: 2.1641x; 2.1641x over previous
import functools

import jax
import jax.numpy as jnp
from jax.experimental import pallas as pl

B, C, N, Q, K = 8, 256, 2048, 64, 1024
NB = 512


def _body(qp_ref, tgt_ref, ze_ref, emb_ref, md_ref, out_ref):
    i = pl.program_id(0)

    emb_v = emb_ref[...]
    emb_sq = jnp.sum(emb_v * emb_v, axis=1)
    ze_v = ze_ref[...]
    ze_sq = jnp.sum(ze_v * ze_v, axis=1)

    acc = jnp.float32(0.0)
    for b in range(B):
        cross = jnp.dot(emb_v, ze_v[b],
                        preferred_element_type=jnp.float32)
        d = emb_sq[:, None] - 2.0 * cross
        acc += jnp.sum(jnp.min(d, axis=0))
    acc += jnp.sum(ze_sq)

    x = qp_ref[...]
    mx = jnp.max(x, axis=1)
    lse = jnp.log(jnp.sum(jnp.exp(x - mx[:, None, :]), axis=1)) + mx
    cidx = jax.lax.broadcasted_iota(jnp.int32, x.shape, 1)
    tv = jnp.sum(jnp.where(cidx == tgt_ref[...][:, None, :], x, 0.0), axis=1)
    acc += jnp.sum(lse - tv)

    md_sum = jnp.sum(md_ref[...])

    @pl.when(i == 0)
    def _():
        out_ref[...] = jnp.zeros_like(out_ref)

    out_ref[0, :] += jnp.broadcast_to(acc, (128,))
    out_ref[1, :] += jnp.broadcast_to(md_sum, (128,))


def kernel(quant_pred, target_wav, ze, emb, min_dist, gamma=0.25):
    tgt = target_wav.astype(jnp.int32)
    out = pl.pallas_call(
        _body,
        grid=(N // NB,),
        in_specs=[
            pl.BlockSpec((B, C, NB), lambda i: (0, 0, i)),
            pl.BlockSpec((B, NB), lambda i: (0, i)),
            pl.BlockSpec((B, Q, NB), lambda i: (0, 0, i)),
            pl.BlockSpec((K, Q), lambda i: (0, 0)),
            pl.BlockSpec((B, NB), lambda i: (0, i)),
        ],
        out_specs=pl.BlockSpec((2, 128), lambda i: (0, 0)),
        out_shape=jax.ShapeDtypeStruct((2, 128), jnp.float32),
    )(quant_pred, tgt, ze, emb, min_dist)
    return out[0, 0] + gamma * out[1, 0]
